# baseline (device time: 21344 ns/iter reference)
import jax
import jax.numpy as jnp
from jax import lax
from jax.experimental import pallas as pl
from jax.experimental.pallas import tpu as pltpu

N_DEV = 4
N_LAYERS = 3


def kernel(x, Win0, Wout0, Win1, Wout1, Win2, Wout2):
    b, d_sh = x.shape
    h_dim = Win0.shape[1]

    def body(x_ref, win0_ref, wout0_ref, win1_ref, wout1_ref, win2_ref,
             wout2_ref, out_ref, comm_ref, send_sems, recv_sems):
        my = lax.axis_index("i")

        barrier_sem = pltpu.get_barrier_semaphore()
        for k in range(1, N_DEV):
            pl.semaphore_signal(
                barrier_sem, inc=1,
                device_id=(lax.rem(my + k, N_DEV),),
                device_id_type=pl.DeviceIdType.MESH,
            )
        pl.semaphore_wait(barrier_sem, N_DEV - 1)

        wins = [win0_ref, win1_ref, win2_ref]
        wouts = [wout0_ref, wout1_ref, wout2_ref]

        x_cur = x_ref[:, :].astype(jnp.bfloat16)
        for layer in range(N_LAYERS):
            w_in = wins[layer][:, :].astype(jnp.bfloat16)
            partial = jnp.dot(x_cur, w_in, preferred_element_type=jnp.float32)
            comm_ref[layer, 0] = partial.astype(jnp.bfloat16)

            rdmas = []
            for k in range(1, N_DEV):
                rdma = pltpu.make_async_remote_copy(
                    src_ref=comm_ref.at[layer, 0],
                    dst_ref=comm_ref.at[layer, k],
                    send_sem=send_sems.at[layer, k - 1],
                    recv_sem=recv_sems.at[layer, k - 1],
                    device_id=(lax.rem(my + k, N_DEV),),
                    device_id_type=pl.DeviceIdType.MESH,
                )
                rdma.start()
                rdmas.append(rdma)
            for rdma in rdmas:
                rdma.wait()

            h = (comm_ref[layer, 0].astype(jnp.float32)
                 + comm_ref[layer, 1].astype(jnp.float32)
                 + comm_ref[layer, 2].astype(jnp.float32)
                 + comm_ref[layer, 3].astype(jnp.float32))
            h = jnp.maximum(h, 0.0).astype(jnp.bfloat16)
            w_out = wouts[layer][:, :].astype(jnp.bfloat16)
            x_cur = jnp.dot(
                h, w_out, preferred_element_type=jnp.float32
            ).astype(jnp.bfloat16)

        out_ref[:, :] = x_cur.astype(jnp.float32)

    return pl.pallas_call(
        body,
        out_shape=jax.ShapeDtypeStruct((b, d_sh), jnp.float32),
        in_specs=[pl.BlockSpec(memory_space=pltpu.VMEM)] * 7,
        out_specs=pl.BlockSpec(memory_space=pltpu.VMEM),
        scratch_shapes=[
            pltpu.VMEM((N_LAYERS, N_DEV, b, h_dim), jnp.bfloat16),
            pltpu.SemaphoreType.DMA((N_LAYERS, N_DEV - 1)),
            pltpu.SemaphoreType.DMA((N_LAYERS, N_DEV - 1)),
        ],
        compiler_params=pltpu.CompilerParams(collective_id=0),
    )(x, Win0, Wout0, Win1, Wout1, Win2, Wout2)


# device time: 21122 ns/iter; 1.0105x vs baseline; 1.0105x over previous
import jax
import jax.numpy as jnp
from jax import lax
from jax.experimental import pallas as pl
from jax.experimental.pallas import tpu as pltpu

N_DEV = 4
N_LAYERS = 3


def kernel(x, Win0, Wout0, Win1, Wout1, Win2, Wout2):
    b, d_sh = x.shape
    h_dim = Win0.shape[1]

    def body(x_ref, win0_ref, wout0_ref, win1_ref, wout1_ref, win2_ref,
             wout2_ref, out_ref, comm_ref, send_sems, recv_sems):
        my = lax.axis_index("i")

        barrier_sem = pltpu.get_barrier_semaphore()
        for k in range(1, N_DEV):
            pl.semaphore_signal(
                barrier_sem, inc=1,
                device_id=(lax.rem(my + k, N_DEV),),
                device_id_type=pl.DeviceIdType.MESH,
            )
        pl.semaphore_wait(barrier_sem, N_DEV - 1)

        def start_sends(layer):
            rdmas = []
            for k in range(1, N_DEV):
                rdma = pltpu.make_async_remote_copy(
                    src_ref=comm_ref.at[layer, 0],
                    dst_ref=comm_ref.at[layer, k],
                    send_sem=send_sems.at[layer, k - 1],
                    recv_sem=recv_sems.at[layer, k - 1],
                    device_id=(lax.rem(my + k, N_DEV),),
                    device_id_type=pl.DeviceIdType.MESH,
                )
                rdma.start()
                rdmas.append(rdma)
            return rdmas

        def reduce_relu(layer, rdmas):
            for rdma in rdmas:
                rdma.wait_recv()
            h = (comm_ref[layer, 0].astype(jnp.float32)
                 + comm_ref[layer, 1].astype(jnp.float32)
                 + comm_ref[layer, 2].astype(jnp.float32)
                 + comm_ref[layer, 3].astype(jnp.float32))
            return jnp.maximum(h, 0.0).astype(jnp.bfloat16)

        x_b = x_ref[:, :].astype(jnp.bfloat16)
        partial = jnp.dot(x_b, win0_ref[:, :].astype(jnp.bfloat16),
                          preferred_element_type=jnp.float32)
        comm_ref[0, 0] = partial.astype(jnp.bfloat16)
        rdmas0 = start_sends(0)

        wf1 = jnp.dot(wout0_ref[:, :].astype(jnp.bfloat16),
                      win1_ref[:, :].astype(jnp.bfloat16),
                      preferred_element_type=jnp.float32).astype(jnp.bfloat16)
        wf2 = jnp.dot(wout1_ref[:, :].astype(jnp.bfloat16),
                      win2_ref[:, :].astype(jnp.bfloat16),
                      preferred_element_type=jnp.float32).astype(jnp.bfloat16)
        wo2 = wout2_ref[:, :].astype(jnp.bfloat16)

        h = reduce_relu(0, rdmas0)
        comm_ref[1, 0] = jnp.dot(
            h, wf1, preferred_element_type=jnp.float32).astype(jnp.bfloat16)
        rdmas1 = start_sends(1)

        h = reduce_relu(1, rdmas1)
        comm_ref[2, 0] = jnp.dot(
            h, wf2, preferred_element_type=jnp.float32).astype(jnp.bfloat16)
        rdmas2 = start_sends(2)

        h = reduce_relu(2, rdmas2)
        out_ref[:, :] = jnp.dot(h, wo2, preferred_element_type=jnp.float32)

        for rdmas in (rdmas0, rdmas1, rdmas2):
            for rdma in rdmas:
                rdma.wait_send()

    return pl.pallas_call(
        body,
        out_shape=jax.ShapeDtypeStruct((b, d_sh), jnp.float32),
        in_specs=[pl.BlockSpec(memory_space=pltpu.VMEM)] * 7,
        out_specs=pl.BlockSpec(memory_space=pltpu.VMEM),
        scratch_shapes=[
            pltpu.VMEM((N_LAYERS, N_DEV, b, h_dim), jnp.bfloat16),
            pltpu.SemaphoreType.DMA((N_LAYERS, N_DEV - 1)),
            pltpu.SemaphoreType.DMA((N_LAYERS, N_DEV - 1)),
        ],
        compiler_params=pltpu.CompilerParams(collective_id=0),
    )(x, Win0, Wout0, Win1, Wout1, Win2, Wout2)


# device time: 20997 ns/iter; 1.0165x vs baseline; 1.0060x over previous
import jax
import jax.numpy as jnp
from jax import lax
from jax.experimental import pallas as pl
from jax.experimental.pallas import tpu as pltpu

N_DEV = 4
N_LAYERS = 3


def kernel(x, Win0, Wout0, Win1, Wout1, Win2, Wout2):
    b, d_sh = x.shape
    h_dim = Win0.shape[1]

    def body(x_ref, win0_ref, wout0_ref, win1_ref, wout1_ref, win2_ref,
             wout2_ref, out_ref, comm_ref, send_sems, recv_sems):
        my = lax.axis_index("i")

        barrier_sem = pltpu.get_barrier_semaphore()
        for k in range(1, N_DEV):
            pl.semaphore_signal(
                barrier_sem, inc=1,
                device_id=(lax.rem(my + k, N_DEV),),
                device_id_type=pl.DeviceIdType.MESH,
            )

        def start_sends(layer):
            rdmas = []
            for k in range(1, N_DEV):
                rdma = pltpu.make_async_remote_copy(
                    src_ref=comm_ref.at[layer, 0],
                    dst_ref=comm_ref.at[layer, k],
                    send_sem=send_sems.at[layer, k - 1],
                    recv_sem=recv_sems.at[layer, k - 1],
                    device_id=(lax.rem(my + k, N_DEV),),
                    device_id_type=pl.DeviceIdType.MESH,
                )
                rdma.start()
                rdmas.append(rdma)
            return rdmas

        def reduce_relu(layer, rdmas):
            for rdma in rdmas:
                rdma.wait_recv()
            h = (comm_ref[layer, 0].astype(jnp.float32)
                 + comm_ref[layer, 1].astype(jnp.float32)
                 + comm_ref[layer, 2].astype(jnp.float32)
                 + comm_ref[layer, 3].astype(jnp.float32))
            return jnp.maximum(h, 0.0).astype(jnp.bfloat16)

        x_b = x_ref[:, :].astype(jnp.bfloat16)
        partial = jnp.dot(x_b, win0_ref[:, :].astype(jnp.bfloat16),
                          preferred_element_type=jnp.float32)
        comm_ref[0, 0] = partial.astype(jnp.bfloat16)
        pl.semaphore_wait(barrier_sem, N_DEV - 1)
        rdmas0 = start_sends(0)

        wf1 = jnp.dot(wout0_ref[:, :].astype(jnp.bfloat16),
                      win1_ref[:, :].astype(jnp.bfloat16),
                      preferred_element_type=jnp.float32).astype(jnp.bfloat16)
        wf2 = jnp.dot(wout1_ref[:, :].astype(jnp.bfloat16),
                      win2_ref[:, :].astype(jnp.bfloat16),
                      preferred_element_type=jnp.float32).astype(jnp.bfloat16)
        wo2 = wout2_ref[:, :].astype(jnp.bfloat16)

        h = reduce_relu(0, rdmas0)
        comm_ref[1, 0] = jnp.dot(
            h, wf1, preferred_element_type=jnp.float32).astype(jnp.bfloat16)
        rdmas1 = start_sends(1)

        h = reduce_relu(1, rdmas1)
        comm_ref[2, 0] = jnp.dot(
            h, wf2, preferred_element_type=jnp.float32).astype(jnp.bfloat16)
        rdmas2 = start_sends(2)

        h = reduce_relu(2, rdmas2)
        out_ref[:, :] = jnp.dot(h, wo2, preferred_element_type=jnp.float32)

        for rdmas in (rdmas0, rdmas1, rdmas2):
            for rdma in rdmas:
                rdma.wait_send()

    return pl.pallas_call(
        body,
        out_shape=jax.ShapeDtypeStruct((b, d_sh), jnp.float32),
        in_specs=[pl.BlockSpec(memory_space=pltpu.VMEM)] * 7,
        out_specs=pl.BlockSpec(memory_space=pltpu.VMEM),
        scratch_shapes=[
            pltpu.VMEM((N_LAYERS, N_DEV, b, h_dim), jnp.bfloat16),
            pltpu.SemaphoreType.DMA((N_LAYERS, N_DEV - 1)),
            pltpu.SemaphoreType.DMA((N_LAYERS, N_DEV - 1)),
        ],
        compiler_params=pltpu.CompilerParams(collective_id=0),
    )(x, Win0, Wout0, Win1, Wout1, Win2, Wout2)
